# baseline (speedup 1.0000x reference)
import functools

import jax
import jax.numpy as jnp
from jax.experimental import pallas as pl
from jax.experimental.pallas import tpu as pltpu

H = 8
LN_EPS = 1e-5


def _attn_kernel(
    x_ref,
    mask_ref,
    q_in_ref,
    wq_ref,
    bq_ref,
    wk_ref,
    bk_ref,
    wv_ref,
    bv_ref,
    wo_ref,
    bo_ref,
    ln_g_ref,
    ln_b_ref,
    outc_ref,
    outa_ref,
    q_scr,
    xbf_scr,
    k_scr,
    v_scr,
    s_scr,
    m_scr,
    l_scr,
    cat_scr,
    *,
    nblk: int,
    bn: int,
    hd: int,
):
    nb = pl.program_id(1)

    @pl.when(nb == 0)
    def _prologue():
        q = jnp.dot(q_in_ref[...], wq_ref[...],
                    preferred_element_type=jnp.float32)
        q = (q + bq_ref[...]) * (1.0 / (hd ** 0.5))
        q_scr[...] = q.astype(jnp.bfloat16)

    xbf_scr[...] = x_ref[0].astype(jnp.bfloat16)
    off = pl.multiple_of(nb * bn, bn)
    k_scr[...] = (jnp.dot(xbf_scr[...], wk_ref[...],
                          preferred_element_type=jnp.float32)
                  + bk_ref[...]).astype(jnp.bfloat16)

    neg = mask_ref[0] * 1e30
    for h in range(H):
        sh = jax.lax.dot_general(
            q_scr[:, h * hd:(h + 1) * hd],
            k_scr[:, h * hd:(h + 1) * hd],
            (((1,), (1,)), ((), ())),
            preferred_element_type=jnp.float32)
        sh = sh - neg
        m_b = jnp.max(sh, axis=1, keepdims=True)
        p = jnp.exp(sh - m_b)
        s_scr[h, :, pl.ds(off, bn)] = p.astype(jnp.bfloat16)
        m_scr[h * nblk + nb] = m_b
        l_scr[h * nblk + nb] = jnp.sum(p, axis=1, keepdims=True)

    v_scr[pl.ds(off, bn), :] = (jnp.dot(xbf_scr[...], wv_ref[...],
                                        preferred_element_type=jnp.float32)
                                + bv_ref[...]).astype(jnp.bfloat16)

    @pl.when(nb == nblk - 1)
    def _finalize():
        nq = s_scr.shape[1]
        fs = []
        for h in range(H):
            ms = [m_scr[h * nblk + c] for c in range(nblk)]
            m_fin = ms[0]
            for c in range(1, nblk):
                m_fin = jnp.maximum(m_fin, ms[c])
            ws = [jnp.exp(ms[c] - m_fin) for c in range(nblk)]
            l_fin = ws[0] * l_scr[h * nblk]
            for c in range(1, nblk):
                l_fin = l_fin + ws[c] * l_scr[h * nblk + c]
            inv = 1.0 / l_fin
            fs.append([ws[c] * inv for c in range(nblk)])
        for h in range(H):
            for c in range(nblk):
                m_scr[h * nblk + c] = fs[h][c]
        cat_scr[...] = jnp.zeros_like(cat_scr)
        ch_w = 512
        blk_per = bn // ch_w

        def chunk_body(c, carry):
            off_c = pl.multiple_of(c * ch_w, ch_w)
            blk = c // blk_per
            attn_sum = None
            for h in range(H):
                f_ch = m_scr[h * nblk + blk]
                attn_ch = (s_scr[h, :, pl.ds(off_c, ch_w)].astype(jnp.float32)
                           * f_ch)
                attn_sum = attn_ch if attn_sum is None else attn_sum + attn_ch
                hs = slice(h * hd, (h + 1) * hd)
                cat_scr[:, hs] = cat_scr[:, hs] + jnp.dot(
                    attn_ch.astype(jnp.bfloat16),
                    v_scr[pl.ds(off_c, ch_w), hs],
                    preferred_element_type=jnp.float32)
            outa_ref[0, :, pl.ds(off_c, ch_w)] = attn_sum * (1.0 / H)
            return carry

        jax.lax.fori_loop(0, nblk * bn // ch_w, chunk_body, 0)
        c = jnp.dot(cat_scr[...].astype(jnp.bfloat16), wo_ref[...],
                    preferred_element_type=jnp.float32) + bo_ref[...]
        mu = jnp.mean(c, axis=1, keepdims=True)
        d = c - mu
        var = jnp.mean(d * d, axis=1, keepdims=True)
        outc_ref[0] = (d * jax.lax.rsqrt(var + LN_EPS) * ln_g_ref[...]
                       + ln_b_ref[...])


def kernel(node_embeddings, node_mask, queries, Wq, bq, Wk, bk, Wv, bv,
           Wo, bo, ln_g, ln_b):
    B, N, d_in = node_embeddings.shape
    nq, d_out = queries.shape
    hd = d_out // H
    bn = min(N, 1024)
    nblk = N // bn

    mask_f = node_mask.astype(jnp.float32).reshape(B * nblk, 1, bn)

    grid = (B, nblk)
    out_shapes = (
        jax.ShapeDtypeStruct((B, nq, d_out), jnp.float32),
        jax.ShapeDtypeStruct((B, nq, N), jnp.float32),
    )
    in_specs = [
        pl.BlockSpec((1, bn, d_in), lambda b, j: (b, j, 0)),
        pl.BlockSpec((1, 1, bn), lambda b, j, _nblk=nblk: (b * _nblk + j, 0, 0)),
        pl.BlockSpec((nq, d_out), lambda b, j: (0, 0)),
        pl.BlockSpec((d_out, d_out), lambda b, j: (0, 0)),
        pl.BlockSpec((1, d_out), lambda b, j: (0, 0)),
        pl.BlockSpec((d_in, d_out), lambda b, j: (0, 0)),
        pl.BlockSpec((1, d_out), lambda b, j: (0, 0)),
        pl.BlockSpec((d_in, d_out), lambda b, j: (0, 0)),
        pl.BlockSpec((1, d_out), lambda b, j: (0, 0)),
        pl.BlockSpec((d_out, d_out), lambda b, j: (0, 0)),
        pl.BlockSpec((1, d_out), lambda b, j: (0, 0)),
        pl.BlockSpec((1, d_out), lambda b, j: (0, 0)),
        pl.BlockSpec((1, d_out), lambda b, j: (0, 0)),
    ]
    out_specs = (
        pl.BlockSpec((1, nq, d_out), lambda b, j: (b, 0, 0)),
        pl.BlockSpec((1, nq, N), lambda b, j: (b, 0, 0)),
    )
    scratch_shapes = [
        pltpu.VMEM((nq, d_out), jnp.bfloat16),
        pltpu.VMEM((bn, d_in), jnp.bfloat16),
        pltpu.VMEM((bn, d_out), jnp.bfloat16),
        pltpu.VMEM((N, d_out), jnp.bfloat16),
        pltpu.VMEM((H, nq, N), jnp.bfloat16),
        pltpu.VMEM((H * nblk, nq, 1), jnp.float32),
        pltpu.VMEM((H * nblk, nq, 1), jnp.float32),
        pltpu.VMEM((nq, d_out), jnp.float32),
    ]
    compressed, attn_avg = pl.pallas_call(
        functools.partial(_attn_kernel, nblk=nblk, bn=bn, hd=hd),
        grid=grid,
        in_specs=in_specs,
        out_specs=out_specs,
        out_shape=out_shapes,
        scratch_shapes=scratch_shapes,
        compiler_params=pltpu.CompilerParams(
            dimension_semantics=("parallel", "arbitrary"),
            vmem_limit_bytes=60 * 1024 * 1024,
        ),
        name="attn_graph_compress",
    )(
        node_embeddings, mask_f, queries.astype(jnp.bfloat16),
        Wq.astype(jnp.bfloat16), bq.reshape(1, d_out),
        Wk.astype(jnp.bfloat16), bk.reshape(1, d_out),
        Wv.astype(jnp.bfloat16), bv.reshape(1, d_out),
        Wo.astype(jnp.bfloat16), bo.reshape(1, d_out),
        ln_g.reshape(1, d_out), ln_b.reshape(1, d_out),
    )
    return compressed, attn_avg


# speedup vs baseline: 1.0163x; 1.0163x over previous
import functools

import jax
import jax.numpy as jnp
from jax.experimental import pallas as pl
from jax.experimental.pallas import tpu as pltpu

H = 8
LN_EPS = 1e-5


def _attn_kernel(
    x_ref,
    mask_ref,
    q_in_ref,
    wq_ref,
    bq_ref,
    wk_ref,
    bk_ref,
    wv_ref,
    bv_ref,
    wo_ref,
    bo_ref,
    ln_g_ref,
    ln_b_ref,
    outc_ref,
    outa_ref,
    q_scr,
    xbf_scr,
    k_scr,
    v_scr,
    s_scr,
    m_scr,
    l_scr,
    cat_scr,
    *,
    nblk: int,
    bn: int,
    hd: int,
):
    nb = pl.program_id(1)

    @pl.when(nb == 0)
    def _prologue():
        q = jnp.dot(q_in_ref[...], wq_ref[...],
                    preferred_element_type=jnp.float32)
        q = (q + bq_ref[...]) * (1.0 / (hd ** 0.5))
        q_scr[...] = q.astype(jnp.bfloat16)
        outa_ref[...] = jnp.zeros_like(outa_ref)

    xbf_scr[...] = x_ref[0].astype(jnp.bfloat16)
    off = pl.multiple_of(nb * bn, bn)
    k_scr[...] = (jnp.dot(xbf_scr[...], wk_ref[...],
                          preferred_element_type=jnp.float32)
                  + bk_ref[...]).astype(jnp.bfloat16)

    neg = mask_ref[0] * 1e30
    for h in range(H):
        sh = jax.lax.dot_general(
            q_scr[:, h * hd:(h + 1) * hd],
            k_scr[:, h * hd:(h + 1) * hd],
            (((1,), (1,)), ((), ())),
            preferred_element_type=jnp.float32)
        sh = sh - neg
        m_b = jnp.max(sh, axis=1, keepdims=True)
        p = jnp.exp(sh - m_b)
        s_scr[h, :, pl.ds(off, bn)] = p.astype(jnp.bfloat16)
        m_scr[h * nblk + nb] = m_b
        l_scr[h * nblk + nb] = jnp.sum(p, axis=1, keepdims=True)

    v_scr[pl.ds(off, bn), :] = (jnp.dot(xbf_scr[...], wv_ref[...],
                                        preferred_element_type=jnp.float32)
                                + bv_ref[...]).astype(jnp.bfloat16)

    @pl.when(nb == nblk - 1)
    def _finalize():
        nq = s_scr.shape[1]

        def head_body(h, carry):
            hof = pl.multiple_of(h * hd, hd)
            sh_ref = s_scr.at[h]
            ms = [m_scr[h * nblk + c] for c in range(nblk)]
            m_fin = ms[0]
            for c in range(1, nblk):
                m_fin = jnp.maximum(m_fin, ms[c])
            ws = [jnp.exp(ms[c] - m_fin) for c in range(nblk)]
            l_fin = ws[0] * l_scr[h * nblk]
            for c in range(1, nblk):
                l_fin = l_fin + ws[c] * l_scr[h * nblk + c]
            inv = 1.0 / l_fin
            acc = jnp.zeros((nq, hd), jnp.float32)
            for c in range(nblk):
                sl = slice(c * bn, (c + 1) * bn)
                f_c = ws[c] * inv
                attn_c = sh_ref[:, sl].astype(jnp.float32) * f_c
                outa_ref[0, :, sl] = outa_ref[0, :, sl] + attn_c * (1.0 / H)
                acc = acc + jnp.dot(attn_c.astype(jnp.bfloat16),
                                    v_scr[sl, pl.ds(hof, hd)],
                                    preferred_element_type=jnp.float32)
            cat_scr[:, pl.ds(hof, hd)] = acc
            return carry

        jax.lax.fori_loop(0, H, head_body, 0)
        c = jnp.dot(cat_scr[...].astype(jnp.bfloat16), wo_ref[...],
                    preferred_element_type=jnp.float32) + bo_ref[...]
        mu = jnp.mean(c, axis=1, keepdims=True)
        d = c - mu
        var = jnp.mean(d * d, axis=1, keepdims=True)
        outc_ref[0] = (d * jax.lax.rsqrt(var + LN_EPS) * ln_g_ref[...]
                       + ln_b_ref[...])


def kernel(node_embeddings, node_mask, queries, Wq, bq, Wk, bk, Wv, bv,
           Wo, bo, ln_g, ln_b):
    B, N, d_in = node_embeddings.shape
    nq, d_out = queries.shape
    hd = d_out // H
    bn = min(N, 1024)
    nblk = N // bn

    mask_f = node_mask.astype(jnp.float32).reshape(B * nblk, 1, bn)

    grid = (B, nblk)
    out_shapes = (
        jax.ShapeDtypeStruct((B, nq, d_out), jnp.float32),
        jax.ShapeDtypeStruct((B, nq, N), jnp.float32),
    )
    in_specs = [
        pl.BlockSpec((1, bn, d_in), lambda b, j: (b, j, 0)),
        pl.BlockSpec((1, 1, bn), lambda b, j, _nblk=nblk: (b * _nblk + j, 0, 0)),
        pl.BlockSpec((nq, d_out), lambda b, j: (0, 0)),
        pl.BlockSpec((d_out, d_out), lambda b, j: (0, 0)),
        pl.BlockSpec((1, d_out), lambda b, j: (0, 0)),
        pl.BlockSpec((d_in, d_out), lambda b, j: (0, 0)),
        pl.BlockSpec((1, d_out), lambda b, j: (0, 0)),
        pl.BlockSpec((d_in, d_out), lambda b, j: (0, 0)),
        pl.BlockSpec((1, d_out), lambda b, j: (0, 0)),
        pl.BlockSpec((d_out, d_out), lambda b, j: (0, 0)),
        pl.BlockSpec((1, d_out), lambda b, j: (0, 0)),
        pl.BlockSpec((1, d_out), lambda b, j: (0, 0)),
        pl.BlockSpec((1, d_out), lambda b, j: (0, 0)),
    ]
    out_specs = (
        pl.BlockSpec((1, nq, d_out), lambda b, j: (b, 0, 0)),
        pl.BlockSpec((1, nq, N), lambda b, j: (b, 0, 0)),
    )
    scratch_shapes = [
        pltpu.VMEM((nq, d_out), jnp.bfloat16),
        pltpu.VMEM((bn, d_in), jnp.bfloat16),
        pltpu.VMEM((bn, d_out), jnp.bfloat16),
        pltpu.VMEM((N, d_out), jnp.bfloat16),
        pltpu.VMEM((H, nq, N), jnp.bfloat16),
        pltpu.VMEM((H * nblk, nq, 1), jnp.float32),
        pltpu.VMEM((H * nblk, nq, 1), jnp.float32),
        pltpu.VMEM((nq, d_out), jnp.float32),
    ]
    compressed, attn_avg = pl.pallas_call(
        functools.partial(_attn_kernel, nblk=nblk, bn=bn, hd=hd),
        grid=grid,
        in_specs=in_specs,
        out_specs=out_specs,
        out_shape=out_shapes,
        scratch_shapes=scratch_shapes,
        compiler_params=pltpu.CompilerParams(
            dimension_semantics=("parallel", "arbitrary"),
            vmem_limit_bytes=60 * 1024 * 1024,
        ),
        name="attn_graph_compress",
    )(
        node_embeddings, mask_f, queries.astype(jnp.bfloat16),
        Wq.astype(jnp.bfloat16), bq.reshape(1, d_out),
        Wk.astype(jnp.bfloat16), bk.reshape(1, d_out),
        Wv.astype(jnp.bfloat16), bv.reshape(1, d_out),
        Wo.astype(jnp.bfloat16), bo.reshape(1, d_out),
        ln_g.reshape(1, d_out), ln_b.reshape(1, d_out),
    )
    return compressed, attn_avg


# speedup vs baseline: 1.0278x; 1.0113x over previous
import functools

import jax
import jax.numpy as jnp
from jax.experimental import pallas as pl
from jax.experimental.pallas import tpu as pltpu

H = 8
LN_EPS = 1e-5


def _attn_kernel(
    x_ref,
    mask_ref,
    q_in_ref,
    wq_ref,
    bq_ref,
    wkv_ref,
    bkv_ref,
    wo_ref,
    bo_ref,
    ln_g_ref,
    ln_b_ref,
    outc_ref,
    outa_ref,
    q_scr,
    xbf_scr,
    k_scr,
    v_scr,
    s_scr,
    m_scr,
    l_scr,
    cat_scr,
    *,
    nblk: int,
    bn: int,
    hd: int,
):
    nb = pl.program_id(1)

    @pl.when(nb == 0)
    def _prologue():
        q = jnp.dot(q_in_ref[...], wq_ref[...],
                    preferred_element_type=jnp.float32)
        q = (q + bq_ref[...]) * (1.0 / (hd ** 0.5))
        q_scr[...] = q.astype(jnp.bfloat16)
        outa_ref[...] = jnp.zeros_like(outa_ref)

    xbf_scr[...] = x_ref[0].astype(jnp.bfloat16)
    off = pl.multiple_of(nb * bn, bn)
    d_out = wq_ref.shape[0]
    kv = jnp.dot(xbf_scr[...], wkv_ref[...],
                 preferred_element_type=jnp.float32) + bkv_ref[...]
    k_scr[...] = kv[:, :d_out].astype(jnp.bfloat16)
    v_scr[pl.ds(off, bn), :] = kv[:, d_out:].astype(jnp.bfloat16)

    neg = mask_ref[0] * 1e30
    for h in range(H):
        sh = jax.lax.dot_general(
            q_scr[:, h * hd:(h + 1) * hd],
            k_scr[:, h * hd:(h + 1) * hd],
            (((1,), (1,)), ((), ())),
            preferred_element_type=jnp.float32)
        sh = sh - neg
        m_b = jnp.max(sh, axis=1, keepdims=True)
        p = jnp.exp(sh - m_b)
        s_scr[h, :, pl.ds(off, bn)] = p.astype(jnp.bfloat16)
        m_scr[h * nblk + nb] = m_b
        l_scr[h * nblk + nb] = jnp.sum(p, axis=1, keepdims=True)

    @pl.when(nb == nblk - 1)
    def _finalize():
        nq = s_scr.shape[1]

        def head_body(h, carry):
            hof = pl.multiple_of(h * hd, hd)
            sh_ref = s_scr.at[h]
            ms = [m_scr[h * nblk + c] for c in range(nblk)]
            m_fin = ms[0]
            for c in range(1, nblk):
                m_fin = jnp.maximum(m_fin, ms[c])
            ws = [jnp.exp(ms[c] - m_fin) for c in range(nblk)]
            l_fin = ws[0] * l_scr[h * nblk]
            for c in range(1, nblk):
                l_fin = l_fin + ws[c] * l_scr[h * nblk + c]
            inv = 1.0 / l_fin
            acc = jnp.zeros((nq, hd), jnp.float32)
            for c in range(nblk):
                sl = slice(c * bn, (c + 1) * bn)
                f_c = ws[c] * inv
                attn_c = sh_ref[:, sl].astype(jnp.float32) * f_c
                outa_ref[0, :, sl] = outa_ref[0, :, sl] + attn_c * (1.0 / H)
                acc = acc + jnp.dot(attn_c.astype(jnp.bfloat16),
                                    v_scr[sl, pl.ds(hof, hd)],
                                    preferred_element_type=jnp.float32)
            cat_scr[:, pl.ds(hof, hd)] = acc
            return carry

        jax.lax.fori_loop(0, H, head_body, 0)
        c = jnp.dot(cat_scr[...].astype(jnp.bfloat16), wo_ref[...],
                    preferred_element_type=jnp.float32) + bo_ref[...]
        mu = jnp.mean(c, axis=1, keepdims=True)
        d = c - mu
        var = jnp.mean(d * d, axis=1, keepdims=True)
        outc_ref[0] = (d * jax.lax.rsqrt(var + LN_EPS) * ln_g_ref[...]
                       + ln_b_ref[...])


def kernel(node_embeddings, node_mask, queries, Wq, bq, Wk, bk, Wv, bv,
           Wo, bo, ln_g, ln_b):
    B, N, d_in = node_embeddings.shape
    nq, d_out = queries.shape
    hd = d_out // H
    bn = min(N, 1024)
    nblk = N // bn

    mask_f = node_mask.astype(jnp.float32).reshape(B * nblk, 1, bn)

    grid = (B, nblk)
    out_shapes = (
        jax.ShapeDtypeStruct((B, nq, d_out), jnp.float32),
        jax.ShapeDtypeStruct((B, nq, N), jnp.float32),
    )
    in_specs = [
        pl.BlockSpec((1, bn, d_in), lambda b, j: (b, j, 0)),
        pl.BlockSpec((1, 1, bn), lambda b, j, _nblk=nblk: (b * _nblk + j, 0, 0)),
        pl.BlockSpec((nq, d_out), lambda b, j: (0, 0)),
        pl.BlockSpec((d_out, d_out), lambda b, j: (0, 0)),
        pl.BlockSpec((1, d_out), lambda b, j: (0, 0)),
        pl.BlockSpec((d_in, 2 * d_out), lambda b, j: (0, 0)),
        pl.BlockSpec((1, 2 * d_out), lambda b, j: (0, 0)),
        pl.BlockSpec((d_out, d_out), lambda b, j: (0, 0)),
        pl.BlockSpec((1, d_out), lambda b, j: (0, 0)),
        pl.BlockSpec((1, d_out), lambda b, j: (0, 0)),
        pl.BlockSpec((1, d_out), lambda b, j: (0, 0)),
    ]
    out_specs = (
        pl.BlockSpec((1, nq, d_out), lambda b, j: (b, 0, 0)),
        pl.BlockSpec((1, nq, N), lambda b, j: (b, 0, 0)),
    )
    scratch_shapes = [
        pltpu.VMEM((nq, d_out), jnp.bfloat16),
        pltpu.VMEM((bn, d_in), jnp.bfloat16),
        pltpu.VMEM((bn, d_out), jnp.bfloat16),
        pltpu.VMEM((N, d_out), jnp.bfloat16),
        pltpu.VMEM((H, nq, N), jnp.bfloat16),
        pltpu.VMEM((H * nblk, nq, 1), jnp.float32),
        pltpu.VMEM((H * nblk, nq, 1), jnp.float32),
        pltpu.VMEM((nq, d_out), jnp.float32),
    ]
    compressed, attn_avg = pl.pallas_call(
        functools.partial(_attn_kernel, nblk=nblk, bn=bn, hd=hd),
        grid=grid,
        in_specs=in_specs,
        out_specs=out_specs,
        out_shape=out_shapes,
        scratch_shapes=scratch_shapes,
        compiler_params=pltpu.CompilerParams(
            dimension_semantics=("parallel", "arbitrary"),
            vmem_limit_bytes=60 * 1024 * 1024,
        ),
        name="attn_graph_compress",
    )(
        node_embeddings, mask_f, queries.astype(jnp.bfloat16),
        Wq.astype(jnp.bfloat16), bq.reshape(1, d_out),
        jnp.concatenate([Wk, Wv], axis=1).astype(jnp.bfloat16),
        jnp.concatenate([bk, bv]).reshape(1, 2 * d_out),
        Wo.astype(jnp.bfloat16), bo.reshape(1, d_out),
        ln_g.reshape(1, d_out), ln_b.reshape(1, d_out),
    )
    return compressed, attn_avg
